# baseline (speedup 1.0000x reference)
import functools

import jax
import jax.numpy as jnp
from jax import lax
from jax.experimental import pallas as pl
from jax.experimental.pallas import tpu as pltpu
from jax.experimental.pallas import tpu_sc as plsc

_NC = 2
_NS = 16
_NW = _NC * _NS


@functools.cache
def _build_sc_copy(n, d, dtype):
    rows = n // _NW
    mesh = plsc.VectorSubcoreMesh(core_axis_name="c", subcore_axis_name="s")

    @functools.partial(
        pl.kernel,
        out_type=jax.ShapeDtypeStruct((n, d), dtype),
        mesh=mesh,
        scratch_types=[pltpu.VMEM((rows, d), dtype)],
    )
    def _sc_copy(x_hbm, o_hbm, buf):
        wid = lax.axis_index("s") * _NC + lax.axis_index("c")
        base = wid * rows
        pltpu.sync_copy(x_hbm.at[pl.ds(base, rows)], buf)
        pltpu.sync_copy(buf, o_hbm.at[pl.ds(base, rows)])

    return _sc_copy


def kernel(x, W):
    n, d = x.shape
    return _build_sc_copy(n, d, x.dtype)(x)


# speedup vs baseline: 2.2224x; 2.2224x over previous
import jax
import jax.numpy as jnp
from jax.experimental import pallas as pl
from jax.experimental.pallas import tpu as pltpu


def _identity_kernel(x_ref, o_ref):
    o_ref[...] = x_ref[...]


def kernel(x, W):
    n, d = x.shape
    flat = jnp.reshape(x, (n * d,))
    out = pl.pallas_call(
        _identity_kernel,
        out_shape=jax.ShapeDtypeStruct((n * d,), x.dtype),
    )(flat)
    return jnp.reshape(out, (n, d))


# speedup vs baseline: 2.8287x; 1.2728x over previous
import jax
import jax.numpy as jnp
from jax.experimental import pallas as pl
from jax.experimental.pallas import tpu as pltpu


def _identity_kernel(x_ref, o_ref):
    o_ref[...] = x_ref[...]


def kernel(x, W):
    n, d = x.shape
    blk = n // 2
    return pl.pallas_call(
        _identity_kernel,
        grid=(2,),
        in_specs=[pl.BlockSpec((blk, d), lambda i: (i, 0))],
        out_specs=pl.BlockSpec((blk, d), lambda i: (i, 0)),
        out_shape=jax.ShapeDtypeStruct(x.shape, x.dtype),
    )(x)
